# Initial kernel scaffold; baseline (speedup 1.0000x reference)
import jax
import jax.numpy as jnp
from jax.experimental import pallas as pl


def kernel(x, edge_index, edge_norm, edge_type, basis, comp, root, bias_rgcn, gc_w1, gc_w2, gc_b, lin_w, lin_b, fc_w, fc_b):
    raise NotImplementedError("write your pallas kernel here")


# speedup vs baseline: 10.2690x; 10.2690x over previous
import functools

import jax
import jax.numpy as jnp
from jax import lax
from jax.experimental import pallas as pl
from jax.experimental.pallas import tpu as pltpu
from jax.experimental.pallas import tpu_sc as plsc

NC = 2
NS = 16
LANES = 16
CH = 128


def _wall_body(comp_ref, basis_ref, root_ref, out_ref):
  r = pl.program_id(0)
  nrel = comp_ref.shape[0]

  @pl.when(r < nrel)
  def _():
    rc = jnp.minimum(r, nrel - 1)
    acc = jnp.zeros(out_ref.shape[1:], jnp.float32)
    for b in range(basis_ref.shape[0]):
      c = comp_ref[pl.ds(rc, 1), pl.ds(b, 1)]
      acc = acc + c * basis_ref[b]
    out_ref[0] = acc

  @pl.when(r == nrel)
  def _():
    out_ref[0] = root_ref[...]


def _xr_body(x_ref, wall_ref, out_ref):
  hh = out_ref.shape[3]
  for r in range(wall_ref.shape[0]):
    d = jnp.dot(x_ref[...], wall_ref[r], preferred_element_type=jnp.float32)
    out_ref[0, r] = d[:, :hh]
    out_ref[1, r] = d[:, hh:]


def _h1_body(agg_ref, xr0_ref, xr1_ref, bias_ref, out_ref):
  hh = agg_ref.shape[2]
  out_ref[0] = agg_ref[0] + xr0_ref[...] + bias_ref[:, :hh]
  out_ref[1] = agg_ref[1] + xr1_ref[...] + bias_ref[:, hh:]


def _out_body(a2_ref, h1_ref, x_ref, gw1_ref, gw2_ref, gb_ref,
              lw1_ref, lw2_ref, lb_ref, fw_ref, fb_ref, out_ref):
  agg2 = jnp.concatenate([a2_ref[0], a2_ref[1]], axis=1)
  h1 = jnp.concatenate([h1_ref[0], h1_ref[1]], axis=1)
  h2 = (jnp.dot(agg2, gw1_ref[...], preferred_element_type=jnp.float32)
        + jnp.dot(h1, gw2_ref[...], preferred_element_type=jnp.float32)
        + gb_ref[...])
  hid = (jnp.dot(x_ref[...], lw1_ref[...], preferred_element_type=jnp.float32)
         + jnp.dot(h2, lw2_ref[...], preferred_element_type=jnp.float32)
         + lb_ref[...])
  hid = jnp.maximum(hid, 0.0)
  logits = jnp.dot(hid, fw_ref[...], preferred_element_type=jnp.float32) + fb_ref[...]
  m = jnp.max(logits, axis=1, keepdims=True)
  lse = m + jnp.log(jnp.sum(jnp.exp(logits - m), axis=1, keepdims=True))
  out_ref[...] = logits - lse


def _edge_phase_a(table, src2, typ2, dst2, nrm2, np_rows, nch, hh):
  rpt = np_rows // NS
  nflush = rpt // CH
  half_rows = table.shape[0] // NC
  mesh = plsc.VectorSubcoreMesh(core_axis_name="c", subcore_axis_name="s",
                                num_cores=NC, num_subcores=NS)

  @functools.partial(
      pl.kernel,
      out_type=jax.ShapeDtypeStruct((NC, np_rows, hh), jnp.float32),
      mesh=mesh,
      scratch_types=[
          pltpu.VMEM((nch, CH), jnp.int32),
          pltpu.VMEM((nch, CH), jnp.int32),
          pltpu.VMEM((nch, CH), jnp.int32),
          pltpu.VMEM((nch, CH), jnp.float32),
          pltpu.VMEM((CH, hh), jnp.float32),
          pltpu.VMEM_SHARED((np_rows, hh), jnp.float32),
          pltpu.SemaphoreType.DMA,
      ],
      compiler_params=pltpu.CompilerParams(use_tc_tiling_on_sc=False),
  )
  def k(table_hbm, src_hbm, typ_hbm, dst_hbm, nrm_hbm, out_hbm,
        srcb, typb, dstb, nrmb, rowsb, acc, gsem):
    cid = lax.axis_index("c")
    sid = lax.axis_index("s")
    pltpu.sync_copy(src_hbm.at[sid], srcb)
    pltpu.sync_copy(typ_hbm.at[sid], typb)
    pltpu.sync_copy(dst_hbm.at[sid], dstb)
    pltpu.sync_copy(nrm_hbm.at[sid], nrmb)

    def zrow(i, carry):
      for g in range(hh // LANES):
        rowsb[i, pl.ds(g * LANES, LANES)] = jnp.zeros((LANES,), jnp.float32)
      return carry
    lax.fori_loop(0, CH, zrow, 0)
    for f in range(nflush):
      pltpu.sync_copy(rowsb, acc.at[pl.ds(sid * rpt + f * CH, CH)])

    base = cid * half_rows
    def cb(i, carry):
      for g in range(CH // LANES):
        sl = pl.ds(g * LANES, LANES)
        srcb[i, sl] = typb[i, sl] * np_rows + srcb[i, sl] + base
      return carry
    lax.fori_loop(0, nch, cb, 0)

    plsc.subcore_barrier()

    def chunk(j, carry):
      pltpu.async_copy(table_hbm.at[srcb.at[j]], rowsb, gsem).wait()
      for grp in range(CH // LANES):
        nrow = nrmb[j, pl.ds(grp * LANES, LANES)]
        for w in range(LANES):
          nv = nrow.at[jnp.full((LANES,), w, jnp.int32)].get(
              mode="promise_in_bounds")
          i = grp * LANES + w
          for g in range(hh // LANES):
            sl = pl.ds(g * LANES, LANES)
            rowsb[i, sl] = rowsb[i, sl] * nv
      pltpu.sync_copy(rowsb, acc.at[dstb.at[j]], add=True)
      return carry
    lax.fori_loop(0, nch, chunk, 0)

    plsc.subcore_barrier()
    for f in range(nflush):
      sl = pl.ds(sid * rpt + f * CH, CH)
      pltpu.sync_copy(acc.at[sl], out_hbm.at[cid].at[sl])

  return k(table, src2, typ2, dst2, nrm2)


def _edge_phase_b(table, src2, dst2, np_rows, nch, hh):
  rpt = np_rows // NS
  nflush = rpt // CH
  half_rows = table.shape[0] // NC
  mesh = plsc.VectorSubcoreMesh(core_axis_name="c", subcore_axis_name="s",
                                num_cores=NC, num_subcores=NS)

  @functools.partial(
      pl.kernel,
      out_type=jax.ShapeDtypeStruct((NC, np_rows, hh), jnp.float32),
      mesh=mesh,
      scratch_types=[
          pltpu.VMEM((nch, CH), jnp.int32),
          pltpu.VMEM((nch, CH), jnp.int32),
          pltpu.VMEM((CH, hh), jnp.float32),
          pltpu.VMEM_SHARED((np_rows, hh), jnp.float32),
          pltpu.SemaphoreType.DMA,
      ],
      compiler_params=pltpu.CompilerParams(use_tc_tiling_on_sc=False),
  )
  def k(table_hbm, src_hbm, dst_hbm, out_hbm, srcb, dstb, rowsb, acc, gsem):
    cid = lax.axis_index("c")
    sid = lax.axis_index("s")
    pltpu.sync_copy(src_hbm.at[sid], srcb)
    pltpu.sync_copy(dst_hbm.at[sid], dstb)

    def zrow(i, carry):
      for g in range(hh // LANES):
        rowsb[i, pl.ds(g * LANES, LANES)] = jnp.zeros((LANES,), jnp.float32)
      return carry
    lax.fori_loop(0, CH, zrow, 0)
    for f in range(nflush):
      pltpu.sync_copy(rowsb, acc.at[pl.ds(sid * rpt + f * CH, CH)])

    base = cid * half_rows
    def cb(i, carry):
      for g in range(CH // LANES):
        sl = pl.ds(g * LANES, LANES)
        srcb[i, sl] = srcb[i, sl] + base
      return carry
    lax.fori_loop(0, nch, cb, 0)

    plsc.subcore_barrier()

    def chunk(j, carry):
      pltpu.async_copy(table_hbm.at[srcb.at[j]], rowsb, gsem).wait()
      pltpu.sync_copy(rowsb, acc.at[dstb.at[j]], add=True)
      return carry
    lax.fori_loop(0, nch, chunk, 0)

    plsc.subcore_barrier()
    for f in range(nflush):
      sl = pl.ds(sid * rpt + f * CH, CH)
      pltpu.sync_copy(acc.at[sl], out_hbm.at[cid].at[sl])

  return k(table, src2, dst2)


def kernel(x, edge_index, edge_norm, edge_type, basis, comp, root, bias_rgcn,
           gc_w1, gc_w2, gc_b, lin_w, lin_b, fc_w, fc_b):
  f32 = jnp.float32
  N, D = x.shape
  H = root.shape[1]
  HH = H // NC
  R, NB = comp.shape
  C = fc_w.shape[1]
  E = edge_index.shape[1]

  NPAD = ((N + 1 + NS * CH - 1) // (NS * CH)) * (NS * CH)
  NCHW = (E + NS * CH - 1) // (NS * CH)
  EPW = NCHW * CH
  EPAD = NS * EPW

  x_p = jnp.pad(x, ((0, NPAD - N), (0, 0)))
  src_p = jnp.pad(edge_index[0], (0, EPAD - E)).reshape(NS, NCHW, CH)
  dst_p = jnp.pad(edge_index[1], (0, EPAD - E),
                  constant_values=N).reshape(NS, NCHW, CH)
  typ_p = jnp.pad(edge_type, (0, EPAD - E)).reshape(NS, NCHW, CH)
  nrm_p = jnp.pad(edge_norm, (0, EPAD - E)).reshape(NS, NCHW, CH)

  wall = pl.pallas_call(
      _wall_body,
      grid=(R + 1,),
      in_specs=[pl.BlockSpec((R, NB), lambda r: (0, 0)),
                pl.BlockSpec((NB, D, H), lambda r: (0, 0, 0)),
                pl.BlockSpec((D, H), lambda r: (0, 0))],
      out_specs=pl.BlockSpec((1, D, H), lambda r: (r, 0, 0)),
      out_shape=jax.ShapeDtypeStruct((R + 1, D, H), f32),
  )(comp, basis, root)

  BN = 512
  NBLK = NPAD // BN

  xr = pl.pallas_call(
      _xr_body,
      grid=(NBLK,),
      in_specs=[pl.BlockSpec((BN, D), lambda i: (i, 0)),
                pl.BlockSpec((R + 1, D, H), lambda i: (0, 0, 0))],
      out_specs=pl.BlockSpec((NC, R + 1, BN, HH), lambda i: (0, 0, i, 0)),
      out_shape=jax.ShapeDtypeStruct((NC, R + 1, NPAD, HH), f32),
  )(x_p, wall)
  xr_flat = xr.reshape(NC * (R + 1) * NPAD, HH)

  agg = _edge_phase_a(xr_flat, src_p, typ_p, dst_p, nrm_p, NPAD, NCHW, HH)

  h1 = pl.pallas_call(
      _h1_body,
      grid=(NBLK,),
      in_specs=[pl.BlockSpec((NC, BN, HH), lambda i: (0, i, 0)),
                pl.BlockSpec((BN, HH),
                             lambda i, _R=R, _NB=NBLK: (_R * _NB + i, 0)),
                pl.BlockSpec((BN, HH),
                             lambda i, _R=R, _NB=NBLK: ((2 * _R + 1) * _NB + i, 0)),
                pl.BlockSpec((1, H), lambda i: (0, 0))],
      out_specs=pl.BlockSpec((NC, BN, HH), lambda i: (0, i, 0)),
      out_shape=jax.ShapeDtypeStruct((NC, NPAD, HH), f32),
  )(agg, xr_flat, xr_flat, bias_rgcn.reshape(1, H))
  h1_flat = h1.reshape(NC * NPAD, HH)

  agg2 = _edge_phase_b(h1_flat, src_p, dst_p, NPAD, NCHW, HH)

  lw1 = lin_w[:D]
  lw2 = lin_w[D:]
  fw_p = jnp.pad(fc_w, ((0, 0), (0, H - C)))
  fb_p = jnp.concatenate([fc_b, jnp.full((H - C,), -1e30, f32)]).reshape(1, H)
  out = pl.pallas_call(
      _out_body,
      grid=(NBLK,),
      in_specs=[pl.BlockSpec((NC, BN, HH), lambda i: (0, i, 0)),
                pl.BlockSpec((NC, BN, HH), lambda i: (0, i, 0)),
                pl.BlockSpec((BN, D), lambda i: (i, 0)),
                pl.BlockSpec((H, H), lambda i: (0, 0)),
                pl.BlockSpec((H, H), lambda i: (0, 0)),
                pl.BlockSpec((1, H), lambda i: (0, 0)),
                pl.BlockSpec((D, H), lambda i: (0, 0)),
                pl.BlockSpec((H, H), lambda i: (0, 0)),
                pl.BlockSpec((1, H), lambda i: (0, 0)),
                pl.BlockSpec((H, H), lambda i: (0, 0)),
                pl.BlockSpec((1, H), lambda i: (0, 0))],
      out_specs=pl.BlockSpec((BN, H), lambda i: (i, 0)),
      out_shape=jax.ShapeDtypeStruct((NPAD, H), f32),
  )(agg2, h1, x_p, gc_w1, gc_w2, gc_b.reshape(1, H),
    lw1, lw2, lin_b.reshape(1, H), fw_p, fb_p)

  return out[:N, :C]


# speedup vs baseline: 12.3901x; 1.2066x over previous
import functools

import jax
import jax.numpy as jnp
from jax import lax
from jax.experimental import pallas as pl
from jax.experimental.pallas import tpu as pltpu
from jax.experimental.pallas import tpu_sc as plsc

NC = 2
NS = 16
LANES = 16
CH = 128


def _wall_body(comp_ref, basis_ref, root_ref, out_ref):
  r = pl.program_id(0)
  nrel = comp_ref.shape[0]

  @pl.when(r < nrel)
  def _():
    rc = jnp.minimum(r, nrel - 1)
    acc = jnp.zeros(out_ref.shape[1:], jnp.float32)
    for b in range(basis_ref.shape[0]):
      c = comp_ref[pl.ds(rc, 1), pl.ds(b, 1)]
      acc = acc + c * basis_ref[b]
    out_ref[0] = acc

  @pl.when(r == nrel)
  def _():
    out_ref[0] = root_ref[...]


def _xr_body(x_ref, wall_ref, out_ref):
  hh = out_ref.shape[3]
  for r in range(wall_ref.shape[0]):
    d = jnp.dot(x_ref[...], wall_ref[r], preferred_element_type=jnp.float32)
    out_ref[0, r] = d[:, :hh]
    out_ref[1, r] = d[:, hh:]


def _out_body(a2_ref, h1_ref, x_ref, gw1_ref, gw2_ref, gb_ref,
              lw1_ref, lw2_ref, lb_ref, fw_ref, fb_ref, out_ref):
  agg2 = jnp.concatenate([a2_ref[0], a2_ref[1]], axis=1)
  h1 = jnp.concatenate([h1_ref[0], h1_ref[1]], axis=1)
  h2 = (jnp.dot(agg2, gw1_ref[...], preferred_element_type=jnp.float32)
        + jnp.dot(h1, gw2_ref[...], preferred_element_type=jnp.float32)
        + gb_ref[...])
  hid = (jnp.dot(x_ref[...], lw1_ref[...], preferred_element_type=jnp.float32)
         + jnp.dot(h2, lw2_ref[...], preferred_element_type=jnp.float32)
         + lb_ref[...])
  hid = jnp.maximum(hid, 0.0)
  logits = jnp.dot(hid, fw_ref[...], preferred_element_type=jnp.float32) + fb_ref[...]
  m = jnp.max(logits, axis=1, keepdims=True)
  lse = m + jnp.log(jnp.sum(jnp.exp(logits - m), axis=1, keepdims=True))
  out_ref[...] = logits - lse


KB = 2
LEAD = 1


def _zero_buf(rowsb, u):
  def zrow(i, carry):
    for g in range(rowsb.shape[2] // LANES):
      rowsb[u, i, pl.ds(g * LANES, LANES)] = jnp.zeros((LANES,), jnp.float32)
    return carry
  lax.fori_loop(0, CH, zrow, 0)


def _wait_gather(table_hbm, srcb, rowsb, gsem, u):
  pltpu.make_async_copy(table_hbm.at[srcb.at[0]], rowsb.at[u],
                        gsem.at[u]).wait()


def _wait_scatter(rowsb, acc, dstb, ssem, u):
  pltpu.make_async_copy(rowsb.at[u], acc.at[dstb.at[0]], ssem.at[u]).wait()


def _scale_rows(rowsb, nrmb, u, j, hh):
  def grp_body(grp, carry):
    nrow = nrmb[j, pl.ds(grp * LANES, LANES)]
    for w in range(LANES):
      nv = nrow.at[jnp.full((LANES,), w, jnp.int32)].get(
          mode="promise_in_bounds")
      i = grp * LANES + w
      for g in range(hh // LANES):
        sl = pl.ds(g * LANES, LANES)
        rowsb[u, i, sl] = rowsb[u, i, sl] * nv
    return carry
  lax.fori_loop(0, CH // LANES, grp_body, 0)


def _pipelined_chunks(table_hbm, idxb, dstb, rowsb, acc, gsem, ssem, nch,
                      scale_fn):
  def _gather(j, u):
    pltpu.async_copy(table_hbm.at[idxb.at[j]], rowsb.at[u], gsem.at[u])

  def _scatter(j, u):
    if scale_fn is not None:
      scale_fn(u, j)
    pltpu.async_copy(rowsb.at[u], acc.at[dstb.at[j]], ssem.at[u], add=True)

  for u in range(KB):
    _gather(u, u)
    if u >= LEAD:
      j2 = u - LEAD
      _wait_gather(table_hbm, idxb, rowsb, gsem, j2)
      _scatter(j2, j2)

  def body(jo, carry):
    for u in range(KB):
      j = jo * KB + u
      v = (u + KB - LEAD) % KB
      _wait_scatter(rowsb, acc, dstb, ssem, u)
      _gather(j, u)
      _wait_gather(table_hbm, idxb, rowsb, gsem, v)
      _scatter(j - LEAD, v)
    return carry
  lax.fori_loop(1, nch // KB, body, 0)

  for t in range(LEAD):
    j2 = nch - LEAD + t
    v = j2 % KB
    _wait_gather(table_hbm, idxb, rowsb, gsem, v)
    _scatter(j2, v)
  for u in range(KB):
    _wait_scatter(rowsb, acc, dstb, ssem, u)


def _edge_phases(table, src2, typ2, dst2, nrm2, bias2, np_rows, nch, hh):
  rpt = np_rows // NS
  nflush = rpt // CH
  half_rows = table.shape[0] // NC
  nrel = half_rows // np_rows - 1
  mesh = plsc.VectorSubcoreMesh(core_axis_name="c", subcore_axis_name="s",
                                num_cores=NC, num_subcores=NS)

  @functools.partial(
      pl.kernel,
      out_type=(jax.ShapeDtypeStruct((NC * np_rows, hh), jnp.float32),
                jax.ShapeDtypeStruct((NC * np_rows, hh), jnp.float32)),
      mesh=mesh,
      scratch_types=[
          pltpu.VMEM((nch, CH), jnp.int32),
          pltpu.VMEM((nch, CH), jnp.int32),
          pltpu.VMEM((nch, CH), jnp.float32),
          pltpu.VMEM((KB, CH, hh), jnp.float32),
          pltpu.VMEM((hh,), jnp.float32),
          pltpu.VMEM_SHARED((np_rows, hh), jnp.float32),
          pltpu.SemaphoreType.DMA((KB,)),
          pltpu.SemaphoreType.DMA((KB,)),
      ],
      compiler_params=pltpu.CompilerParams(use_tc_tiling_on_sc=False),
  )
  def k(table_hbm, src_hbm, typ_hbm, dst_hbm, nrm_hbm, bias_hbm,
        h1_hbm, out_hbm,
        combb, dstb, nrmb, rowsb, biasb, acc, gsem, ssem):
    cid = lax.axis_index("c")
    sid = lax.axis_index("s")
    base_a = cid * half_rows
    base_b = cid * np_rows

    pltpu.sync_copy(typ_hbm.at[sid], combb)
    pltpu.sync_copy(src_hbm.at[sid], dstb)
    def cb_a(i, carry):
      for g in range(CH // LANES):
        sl = pl.ds(g * LANES, LANES)
        combb[i, sl] = combb[i, sl] * np_rows + dstb[i, sl] + base_a
      return carry
    lax.fori_loop(0, nch, cb_a, 0)
    pltpu.sync_copy(dst_hbm.at[sid], dstb)
    pltpu.sync_copy(nrm_hbm.at[sid], nrmb)
    pltpu.sync_copy(bias_hbm.at[cid], biasb)

    _zero_buf(rowsb, 0)
    for f in range(nflush):
      pltpu.sync_copy(rowsb.at[0], acc.at[pl.ds(sid * rpt + f * CH, CH)])

    plsc.subcore_barrier()

    scale = lambda u, j: _scale_rows(rowsb, nrmb, u, j, hh)
    _pipelined_chunks(table_hbm, combb, dstb, rowsb, acc, gsem, ssem, nch,
                      scale)

    plsc.subcore_barrier()

    xroot_base = (cid * (nrel + 1) + nrel) * np_rows
    for f in range(nflush):
      row0 = sid * rpt + f * CH
      pltpu.sync_copy(acc.at[pl.ds(row0, CH)], rowsb.at[0])
      pltpu.sync_copy(table_hbm.at[pl.ds(xroot_base + row0, CH)],
                      rowsb.at[1])
      def hrow(i, carry):
        for g in range(hh // LANES):
          sl = pl.ds(g * LANES, LANES)
          rowsb[0, i, sl] = rowsb[0, i, sl] + rowsb[1, i, sl] + biasb[sl]
        return carry
      lax.fori_loop(0, CH, hrow, 0)
      pltpu.sync_copy(rowsb.at[0], h1_hbm.at[pl.ds(base_b + row0, CH)])
    _zero_buf(rowsb, 1)
    for f in range(nflush):
      pltpu.sync_copy(rowsb.at[1], acc.at[pl.ds(sid * rpt + f * CH, CH)])

    pltpu.sync_copy(src_hbm.at[sid], combb)
    def cb_b(i, carry):
      for g in range(CH // LANES):
        sl = pl.ds(g * LANES, LANES)
        combb[i, sl] = combb[i, sl] + base_b
      return carry
    lax.fori_loop(0, nch, cb_b, 0)

    plsc.subcore_barrier()

    _pipelined_chunks(h1_hbm, combb, dstb, rowsb, acc, gsem, ssem, nch,
                      None)

    plsc.subcore_barrier()
    for f in range(nflush):
      row0 = sid * rpt + f * CH
      pltpu.sync_copy(acc.at[pl.ds(row0, CH)],
                      out_hbm.at[pl.ds(base_b + row0, CH)])

  return k(table, src2, typ2, dst2, nrm2, bias2)


def kernel(x, edge_index, edge_norm, edge_type, basis, comp, root, bias_rgcn,
           gc_w1, gc_w2, gc_b, lin_w, lin_b, fc_w, fc_b):
  f32 = jnp.float32
  N, D = x.shape
  H = root.shape[1]
  HH = H // NC
  R, NB = comp.shape
  C = fc_w.shape[1]
  E = edge_index.shape[1]

  NPAD = ((N + 1 + NS * CH - 1) // (NS * CH)) * (NS * CH)
  NCHW = (E + NS * CH - 1) // (NS * CH)
  NCHW = ((NCHW + KB - 1) // KB) * KB
  EPW = NCHW * CH
  EPAD = NS * EPW

  x_p = jnp.pad(x, ((0, NPAD - N), (0, 0)))
  src_p = jnp.pad(edge_index[0], (0, EPAD - E)).reshape(NS, NCHW, CH)
  dst_p = jnp.pad(edge_index[1], (0, EPAD - E),
                  constant_values=N).reshape(NS, NCHW, CH)
  typ_p = jnp.pad(edge_type, (0, EPAD - E)).reshape(NS, NCHW, CH)
  nrm_p = jnp.pad(edge_norm, (0, EPAD - E)).reshape(NS, NCHW, CH)

  wall = pl.pallas_call(
      _wall_body,
      grid=(R + 1,),
      in_specs=[pl.BlockSpec((R, NB), lambda r: (0, 0)),
                pl.BlockSpec((NB, D, H), lambda r: (0, 0, 0)),
                pl.BlockSpec((D, H), lambda r: (0, 0))],
      out_specs=pl.BlockSpec((1, D, H), lambda r: (r, 0, 0)),
      out_shape=jax.ShapeDtypeStruct((R + 1, D, H), f32),
  )(comp, basis, root)

  BN = 512
  NBLK = NPAD // BN

  xr = pl.pallas_call(
      _xr_body,
      grid=(NBLK,),
      in_specs=[pl.BlockSpec((BN, D), lambda i: (i, 0)),
                pl.BlockSpec((R + 1, D, H), lambda i: (0, 0, 0))],
      out_specs=pl.BlockSpec((NC, R + 1, BN, HH), lambda i: (0, 0, i, 0)),
      out_shape=jax.ShapeDtypeStruct((NC, R + 1, NPAD, HH), f32),
  )(x_p, wall)
  xr_flat = xr.reshape(NC * (R + 1) * NPAD, HH)

  h1_flat, agg2_flat = _edge_phases(xr_flat, src_p, typ_p, dst_p, nrm_p,
                                    bias_rgcn.reshape(NC, HH),
                                    NPAD, NCHW, HH)
  h1 = h1_flat.reshape(NC, NPAD, HH)
  agg2 = agg2_flat.reshape(NC, NPAD, HH)

  lw1 = lin_w[:D]
  lw2 = lin_w[D:]
  fw_p = jnp.pad(fc_w, ((0, 0), (0, H - C)))
  fb_p = jnp.concatenate([fc_b, jnp.full((H - C,), -1e30, f32)]).reshape(1, H)
  out = pl.pallas_call(
      _out_body,
      grid=(NBLK,),
      in_specs=[pl.BlockSpec((NC, BN, HH), lambda i: (0, i, 0)),
                pl.BlockSpec((NC, BN, HH), lambda i: (0, i, 0)),
                pl.BlockSpec((BN, D), lambda i: (i, 0)),
                pl.BlockSpec((H, H), lambda i: (0, 0)),
                pl.BlockSpec((H, H), lambda i: (0, 0)),
                pl.BlockSpec((1, H), lambda i: (0, 0)),
                pl.BlockSpec((D, H), lambda i: (0, 0)),
                pl.BlockSpec((H, H), lambda i: (0, 0)),
                pl.BlockSpec((1, H), lambda i: (0, 0)),
                pl.BlockSpec((H, H), lambda i: (0, 0)),
                pl.BlockSpec((1, H), lambda i: (0, 0))],
      out_specs=pl.BlockSpec((BN, H), lambda i: (i, 0)),
      out_shape=jax.ShapeDtypeStruct((NPAD, H), f32),
  )(agg2, h1, x_p, gc_w1, gc_w2, gc_b.reshape(1, H),
    lw1, lw2, lin_b.reshape(1, H), fw_p, fb_p)

  return out[:N, :C]
